# baseline (speedup 1.0000x reference)
import jax
import jax.numpy as jnp
from jax.experimental import pallas as pl
from jax.experimental.pallas import tpu as pltpu

_K = 300


def _select_kernel(logits_ref, coords_ref, idx_ref, tlog_ref, tcrd_ref):
    n = logits_ref.shape[1]
    scores = jnp.max(logits_ref[...], axis=2)
    iota = jax.lax.broadcasted_iota(jnp.int32, (1, n), 1)

    def body(k, s):
        m = jnp.max(s)
        i = jnp.min(jnp.where(s == m, iota, n))
        idx_ref[0, 0, k] = i
        tlog_ref[0, pl.ds(k, 1), :] = logits_ref[0, pl.ds(i, 1), :]
        tcrd_ref[0, pl.ds(k, 1), :] = coords_ref[0, pl.ds(i, 1), :]
        return jnp.where(iota == i, -jnp.inf, s)

    jax.lax.fori_loop(0, _K, body, scores)


def _gather_mem_kernel(idx_ref, mem_ref, out_ref):
    b = pl.program_id(0)

    def body(k, carry):
        i = idx_ref[b, 0, k]
        out_ref[0, pl.ds(k, 1), :] = mem_ref[0, pl.ds(i, 1), :]
        return carry

    jax.lax.fori_loop(0, _K, body, 0)


def kernel(memory, query_class_logits, query_geometries_unactivated):
    B, N, D = memory.shape
    C = query_class_logits.shape[-1]

    idx, tlog, tcrd = pl.pallas_call(
        _select_kernel,
        grid=(B,),
        in_specs=[
            pl.BlockSpec((1, N, C), lambda b: (b, 0, 0)),
            pl.BlockSpec((1, N, 4), lambda b: (b, 0, 0)),
        ],
        out_specs=[
            pl.BlockSpec((1, 1, _K), lambda b: (b, 0, 0),
                         memory_space=pltpu.SMEM),
            pl.BlockSpec((1, _K, C), lambda b: (b, 0, 0)),
            pl.BlockSpec((1, _K, 4), lambda b: (b, 0, 0)),
        ],
        out_shape=[
            jax.ShapeDtypeStruct((B, 1, _K), jnp.int32),
            jax.ShapeDtypeStruct((B, _K, C), jnp.float32),
            jax.ShapeDtypeStruct((B, _K, 4), jnp.float32),
        ],
    )(query_class_logits, query_geometries_unactivated)

    tmem = pl.pallas_call(
        _gather_mem_kernel,
        grid_spec=pltpu.PrefetchScalarGridSpec(
            num_scalar_prefetch=1,
            grid=(B,),
            in_specs=[pl.BlockSpec((1, N, D), lambda b, idx: (b, 0, 0))],
            out_specs=pl.BlockSpec((1, _K, D), lambda b, idx: (b, 0, 0)),
        ),
        out_shape=jax.ShapeDtypeStruct((B, _K, D), jnp.float32),
    )(idx, memory)

    return tmem, tlog, tcrd


# speedup vs baseline: 1.6073x; 1.6073x over previous
import jax
import jax.numpy as jnp
from jax.experimental import pallas as pl
from jax.experimental.pallas import tpu as pltpu

_K = 300
_KPAD = 384
_LANES = 128


def _select_kernel(logits_ref, idx_ref):
    n = logits_ref.shape[1]
    rows = (n + _LANES - 1) // _LANES
    pad = rows * _LANES - n

    s = jnp.max(logits_ref[...], axis=2)
    s = jnp.concatenate(
        [s, jnp.full((1, pad), -jnp.inf, jnp.float32)], axis=1)
    s = s.reshape(rows, _LANES)

    n2d = (jax.lax.broadcasted_iota(jnp.int32, (rows, _LANES), 0) * _LANES
           + jax.lax.broadcasted_iota(jnp.int32, (rows, _LANES), 1))
    lane_k = jax.lax.broadcasted_iota(jnp.int32, (1, _KPAD), 1)

    def body(k, carry):
        s, acc = carry
        m = jnp.max(s, axis=(0, 1), keepdims=True)
        ivec = jnp.min(jnp.where(s >= m, n2d, n + pad),
                       axis=(0, 1), keepdims=True)
        s = jnp.where(n2d == ivec, -jnp.inf, s)
        acc = jnp.where(lane_k == k, ivec, acc)
        return s, acc

    _, acc = jax.lax.fori_loop(
        0, _K, body, (s, jnp.zeros((1, _KPAD), jnp.int32)))
    idx_ref[0] = acc[:, :_K]


def _gather_lc_kernel(idx_ref, logits_ref, coords_ref, tlog_ref, tcrd_ref):
    b = pl.program_id(0)

    def body(k, carry):
        i = idx_ref[b, 0, k]
        tlog_ref[0, pl.ds(k, 1), :] = logits_ref[0, pl.ds(i, 1), :]
        tcrd_ref[0, pl.ds(k, 1), :] = coords_ref[0, pl.ds(i, 1), :]
        return carry

    jax.lax.fori_loop(0, _K, body, 0)


def _gather_mem_kernel(idx_ref, mem_ref, out_ref):
    b = pl.program_id(0)

    def body(k, carry):
        i = idx_ref[b, 0, k]
        out_ref[0, pl.ds(k, 1), :] = mem_ref[0, pl.ds(i, 1), :]
        return carry

    jax.lax.fori_loop(0, _K, body, 0)


def kernel(memory, query_class_logits, query_geometries_unactivated):
    B, N, D = memory.shape
    C = query_class_logits.shape[-1]

    idx = pl.pallas_call(
        _select_kernel,
        grid=(B,),
        in_specs=[pl.BlockSpec((1, N, C), lambda b: (b, 0, 0))],
        out_specs=pl.BlockSpec((1, 1, _K), lambda b: (b, 0, 0)),
        out_shape=jax.ShapeDtypeStruct((B, 1, _K), jnp.int32),
        compiler_params=pltpu.CompilerParams(
            dimension_semantics=("parallel",)),
    )(query_class_logits)

    tlog, tcrd = pl.pallas_call(
        _gather_lc_kernel,
        grid_spec=pltpu.PrefetchScalarGridSpec(
            num_scalar_prefetch=1,
            grid=(B,),
            in_specs=[
                pl.BlockSpec((1, N, C), lambda b, idx: (b, 0, 0)),
                pl.BlockSpec((1, N, 4), lambda b, idx: (b, 0, 0)),
            ],
            out_specs=[
                pl.BlockSpec((1, _K, C), lambda b, idx: (b, 0, 0)),
                pl.BlockSpec((1, _K, 4), lambda b, idx: (b, 0, 0)),
            ],
        ),
        out_shape=[
            jax.ShapeDtypeStruct((B, _K, C), jnp.float32),
            jax.ShapeDtypeStruct((B, _K, 4), jnp.float32),
        ],
        compiler_params=pltpu.CompilerParams(
            dimension_semantics=("parallel",)),
    )(idx, query_class_logits, query_geometries_unactivated)

    tmem = pl.pallas_call(
        _gather_mem_kernel,
        grid_spec=pltpu.PrefetchScalarGridSpec(
            num_scalar_prefetch=1,
            grid=(B,),
            in_specs=[pl.BlockSpec((1, N, D), lambda b, idx: (b, 0, 0))],
            out_specs=pl.BlockSpec((1, _K, D), lambda b, idx: (b, 0, 0)),
        ),
        out_shape=jax.ShapeDtypeStruct((B, _K, D), jnp.float32),
        compiler_params=pltpu.CompilerParams(
            dimension_semantics=("parallel",)),
    )(idx, memory)

    return tmem, tlog, tcrd
